# Initial kernel scaffold; baseline (speedup 1.0000x reference)
import jax
import jax.numpy as jnp
from jax.experimental import pallas as pl


def kernel(x, edge1, pos, idx, ei2, emb, gne_w, gne_b, gne_ms, W1, b1, gn1_w, gn1_b, gn1_ms, W2, b2, gn2_w, gn2_b, gn2_ms, W3, b3, gn3_w, gn3_b, gn3_ms, W4, b4, gn4_w, gn4_b, gn4_ms, Wp, bp):
    raise NotImplementedError("write your pallas kernel here")


# speedup vs baseline: 12.6754x; 12.6754x over previous
import functools

import jax
import jax.numpy as jnp
from jax import lax
from jax.experimental import pallas as pl
from jax.experimental.pallas import tpu as pltpu
from jax.experimental.pallas import tpu_sc as plsc

N = 100000
E = 1600000
P = 200000
B = 100000
E2 = 800000
C = 32
RN = 102400
RP = 204800

_MESH = plsc.VectorSubcoreMesh(core_axis_name="c", subcore_axis_name="s")
_SC_PARAMS = pltpu.CompilerParams(use_tc_tiling_on_sc=False)
F32 = jnp.float32
I32 = jnp.int32


def _zero_init_1d(zb, acc, size, sid):
    per = size // 16
    base = sid * per
    for j in range(per // 6400):
        pltpu.sync_copy(zb, acc.at[pl.ds(base + j * 6400, 6400)])


def _copy_1d(acc, out, size, sid):
    per = size // 16
    base = sid * per
    for j in range(per // 6400):
        pltpu.sync_copy(acc.at[pl.ds(base + j * 6400, 6400)],
                        out.at[pl.ds(base + j * 6400, 6400)])


def _count_1d(acc, idx2d, nchunks, ob, ib, sid):
    niter = (nchunks + 15) // 16

    def body(i, carry):
        c = i * 16 + sid

        @pl.when(c < nchunks)
        def _():
            pltpu.sync_copy(idx2d.at[c], ib)
            pltpu.sync_copy(ob, acc.at[ib], add=True)

        return carry

    lax.fori_loop(0, niter, body, 0)


def _sck_deg(dst1_2d, e2d_2d, e2s_2d, zeros6400, ones128):
    @functools.partial(
        pl.kernel, mesh=_MESH, compiler_params=_SC_PARAMS,
        out_type=[jax.ShapeDtypeStruct((RN,), F32),
                  jax.ShapeDtypeStruct((RP,), F32),
                  jax.ShapeDtypeStruct((RP,), F32)],
        scratch_types=[pltpu.VMEM_SHARED((RN,), F32),
                       pltpu.VMEM_SHARED((RP,), F32),
                       pltpu.VMEM_SHARED((RP,), F32),
                       pltpu.VMEM((6400,), F32),
                       pltpu.VMEM((128,), F32),
                       pltpu.VMEM((128,), I32)],
    )
    def k(dst1_ref, e2d_ref, e2s_ref, z_ref, o128_ref,
          deg1_ref, degA_ref, degB_ref,
          acc1, accA, accB, zb, ob, ib):
        cid = lax.axis_index("c")
        sid = lax.axis_index("s")
        pltpu.sync_copy(z_ref, zb)
        pltpu.sync_copy(o128_ref, ob)

        @pl.when(cid == 0)
        def _():
            _zero_init_1d(zb, acc1, RN, sid)

        @pl.when(cid == 1)
        def _():
            _zero_init_1d(zb, accA, RP, sid)
            _zero_init_1d(zb, accB, RP, sid)

        plsc.subcore_barrier()

        @pl.when(cid == 0)
        def _():
            _count_1d(acc1, dst1_ref, E // 128, ob, ib, sid)

        @pl.when(cid == 1)
        def _():
            _count_1d(accA, e2d_ref, E2 // 128, ob, ib, sid)
            _count_1d(accB, e2s_ref, E2 // 128, ob, ib, sid)

        plsc.subcore_barrier()

        @pl.when(cid == 0)
        def _():
            _copy_1d(acc1, deg1_ref, RN, sid)

        @pl.when(cid == 1)
        def _():
            _copy_1d(accA, degA_ref, RP, sid)
            _copy_1d(accB, degB_ref, RP, sid)

    return k(dst1_2d, e2d_2d, e2s_2d, zeros6400, ones128)


def _zero_init_rows(zb, acc, nrows, ch, sid):
    per = nrows // 16
    base = sid * per
    for j in range(per // ch):
        pltpu.sync_copy(zb, acc.at[pl.ds(base + j * ch, ch)])


def _copy_rows(acc, out, nrows, ch, sid):
    per = nrows // 16
    base = sid * per
    for j in range(per // ch):
        pltpu.sync_copy(acc.at[pl.ds(base + j * ch, ch)],
                        out.at[pl.ds(base + j * ch, ch)])


def _edge_scan(tbl, acc, sidx2d, didx2d, nchunks, sv, dv, rows, sem, sid):
    nmfull, tailch = divmod(nchunks, 8)
    nmb = nmfull + (1 if tailch else 0)
    niter = (nmb + 15) // 16

    def macro(m, nch):
        pltpu.sync_copy(sidx2d.at[pl.ds(m * 8, nch)], sv.at[pl.ds(0, nch)])
        pltpu.sync_copy(didx2d.at[pl.ds(m * 8, nch)], dv.at[pl.ds(0, nch)])
        for j in range(nch):
            pltpu.async_copy(tbl.at[sv.at[j]], rows, sem).wait()
            pltpu.sync_copy(rows, acc.at[dv.at[j]], add=True)

    def body(i, carry):
        m = i * 16 + sid

        @pl.when(m < nmfull)
        def _():
            macro(m, 8)

        if tailch:
            @pl.when(m == nmfull)
            def _():
                macro(m, tailch)

        return carry

    lax.fori_loop(0, niter, body, 0)


def _sck_conv(T0, T1, sidx2d, didx2d, zeros640x16):
    V = T0.shape[0]
    nchunks = sidx2d.shape[0]

    @functools.partial(
        pl.kernel, mesh=_MESH, compiler_params=_SC_PARAMS,
        out_type=[jax.ShapeDtypeStruct((V, 16), F32),
                  jax.ShapeDtypeStruct((V, 16), F32)],
        scratch_types=[pltpu.VMEM_SHARED((V, 16), F32),
                       pltpu.VMEM((640, 16), F32),
                       pltpu.VMEM((8, 128), I32),
                       pltpu.VMEM((8, 128), I32),
                       pltpu.VMEM((128, 16), F32),
                       pltpu.SemaphoreType.DMA],
    )
    def k(t0_ref, t1_ref, s_ref, d_ref, z_ref, a0_ref, a1_ref,
          acc, zb, sv, dv, rows, sem):
        cid = lax.axis_index("c")
        sid = lax.axis_index("s")
        pltpu.sync_copy(z_ref, zb)
        _zero_init_rows(zb, acc, V, 640, sid)
        plsc.subcore_barrier()

        @pl.when(cid == 0)
        def _():
            _edge_scan(t0_ref, acc, s_ref, d_ref, nchunks, sv, dv, rows, sem, sid)

        @pl.when(cid == 1)
        def _():
            _edge_scan(t1_ref, acc, s_ref, d_ref, nchunks, sv, dv, rows, sem, sid)

        plsc.subcore_barrier()

        @pl.when(cid == 0)
        def _():
            _copy_rows(acc, a0_ref, V, 640, sid)

        @pl.when(cid == 1)
        def _():
            _copy_rows(acc, a1_ref, V, 640, sid)

    return k(T0, T1, sidx2d, didx2d, zeros640x16)


def _sck_zconv(gA, gB, e2s2d, e2d2d, zeros1280x8):
    nchunks = e2s2d.shape[0]
    out_t = [jax.ShapeDtypeStruct((RP, 8), F32) for _ in range(6)]

    @functools.partial(
        pl.kernel, mesh=_MESH, compiler_params=_SC_PARAMS,
        out_type=out_t,
        scratch_types=[pltpu.VMEM_SHARED((RP, 8), F32),
                       pltpu.VMEM((1280, 8), F32),
                       pltpu.VMEM((8, 128), I32),
                       pltpu.VMEM((8, 128), I32),
                       pltpu.VMEM((128, 8), F32),
                       pltpu.SemaphoreType.DMA],
    )
    def k(ga0, ga1, ga2, gb0, gb1, gb2, es_ref, ed_ref, z_ref,
          oa0, oa1, oa2, ob0, ob1, ob2,
          acc, zb, sv, dv, rows, sem):
        cid = lax.axis_index("c")
        sid = lax.axis_index("s")
        pltpu.sync_copy(z_ref, zb)
        ga = [ga0, ga1, ga2]
        gb = [gb0, gb1, gb2]
        oa = [oa0, oa1, oa2]
        ob = [ob0, ob1, ob2]
        for r in range(3):
            _zero_init_rows(zb, acc, RP, 1280, sid)
            plsc.subcore_barrier()

            @pl.when(cid == 0)
            def _():
                _edge_scan(ga[r], acc, es_ref, ed_ref, nchunks, sv, dv, rows,
                           sem, sid)

            @pl.when(cid == 1)
            def _():
                _edge_scan(gb[r], acc, ed_ref, es_ref, nchunks, sv, dv, rows,
                           sem, sid)

            plsc.subcore_barrier()

            @pl.when(cid == 0)
            def _():
                _copy_rows(acc, oa[r], RP, 1280, sid)

            @pl.when(cid == 1)
            def _():
                _copy_rows(acc, ob[r], RP, 1280, sid)

            if r < 2:
                plsc.subcore_barrier()

    return k(gA[0], gA[1], gA[2], gB[0], gB[1], gB[2], e2s2d, e2d2d,
             zeros1280x8)


def _sck_gather(table, idxs):
    Bn = idxs.shape[0]
    nch = Bn // 128
    niter = nch // 32

    @functools.partial(
        pl.kernel, mesh=_MESH, compiler_params=_SC_PARAMS,
        out_type=jax.ShapeDtypeStruct((Bn, 32), F32),
        scratch_types=[pltpu.VMEM((128,), I32),
                       pltpu.VMEM((128, 32), F32),
                       pltpu.SemaphoreType.DMA],
    )
    def k(tbl, idx_ref, out_ref, iv, rows, sem):
        cid = lax.axis_index("c")
        sid = lax.axis_index("s")
        wid = sid * 2 + cid

        def body(i, carry):
            c = i * 32 + wid
            base = c * 128
            pltpu.sync_copy(idx_ref.at[pl.ds(base, 128)], iv)
            pltpu.async_copy(tbl.at[iv], rows, sem).wait()
            pltpu.sync_copy(rows, out_ref.at[pl.ds(base, 128)])
            return carry

        lax.fori_loop(0, niter, body, 0)

    return k(table, idxs)


def _sck_hadamard(table, idxl, idxr):
    Bn = idxl.shape[0]
    nch = Bn // 128
    niter = nch // 32

    @functools.partial(
        pl.kernel, mesh=_MESH, compiler_params=_SC_PARAMS,
        out_type=[jax.ShapeDtypeStruct((Bn, 32), F32),
                  jax.ShapeDtypeStruct((Bn, 32), F32)],
        scratch_types=[pltpu.VMEM((128,), I32),
                       pltpu.VMEM((128,), I32),
                       pltpu.VMEM((128, 32), F32),
                       pltpu.VMEM((128, 32), F32),
                       pltpu.SemaphoreType.DMA,
                       pltpu.SemaphoreType.DMA],
    )
    def k(tbl, il_ref, ir_ref, zl_ref, zr_ref,
          ivl, ivr, rl, rr, sem1, sem2):
        cid = lax.axis_index("c")
        sid = lax.axis_index("s")
        wid = sid * 2 + cid

        def body(i, carry):
            c = i * 32 + wid
            base = c * 128
            pltpu.sync_copy(il_ref.at[pl.ds(base, 128)], ivl)
            pltpu.sync_copy(ir_ref.at[pl.ds(base, 128)], ivr)
            cp1 = pltpu.async_copy(tbl.at[ivl], rl, sem1)
            cp2 = pltpu.async_copy(tbl.at[ivr], rr, sem2)
            cp1.wait()
            cp2.wait()
            pltpu.sync_copy(rl, zl_ref.at[pl.ds(base, 128)])
            pltpu.sync_copy(rr, zr_ref.at[pl.ds(base, 128)])
            return carry

        lax.fori_loop(0, niter, body, 0)

    return k(table, idxl, idxr)


_BR = 2048


def _rowmask(i, r_real):
    gr = i * _BR + lax.broadcasted_iota(I32, (_BR, C), 0)
    return gr < r_real


def _tck_rsqrt(deg):
    R = deg.shape[0]
    BRX = 4096

    def body(d_ref, o_ref):
        o_ref[...] = lax.rsqrt(d_ref[...] + 1.0)

    return pl.pallas_call(
        body,
        grid=(R // BRX,),
        in_specs=[pl.BlockSpec((BRX,), lambda i: (i,))],
        out_specs=pl.BlockSpec((BRX,), lambda i: (i,)),
        out_shape=jax.ShapeDtypeStruct((R,), F32),
    )(deg)


def _tck_stats(y, r_real):
    R = y.shape[0]

    def body(y_ref, s_ref, q_ref):
        i = pl.program_id(0)

        @pl.when(i == 0)
        def _():
            s_ref[...] = jnp.zeros_like(s_ref)
            q_ref[...] = jnp.zeros_like(q_ref)

        yb = jnp.where(_rowmask(i, r_real), y_ref[...], 0.0)
        s_ref[...] += jnp.sum(yb, axis=0, keepdims=True)
        q_ref[...] += jnp.sum(yb * yb, axis=0, keepdims=True)

    return pl.pallas_call(
        body,
        grid=(R // _BR,),
        in_specs=[pl.BlockSpec((_BR, C), lambda i: (i, 0))],
        out_specs=[pl.BlockSpec((1, C), lambda i: (0, 0)),
                   pl.BlockSpec((1, C), lambda i: (0, 0))],
        out_shape=[jax.ShapeDtypeStruct((1, C), F32),
                   jax.ShapeDtypeStruct((1, C), F32)],
    )(y)


def _gn_block(yb, s, q, w, b, ms, r_real):
    mean = s * (1.0 / r_real)
    m2 = q * (1.0 / r_real)
    mm = ms * mean
    var = m2 - 2.0 * mm * mean + mm * mm
    rstd = lax.rsqrt(var + 1e-5)
    return w * (yb - mm) * rstd + b


def _tck_table(y, S, Q, w2d, b2d, ms2d, W, dinvb, relu, r_real):
    R = y.shape[0]

    def body(y_ref, s_ref, q_ref, w_ref, b_ref, m_ref, W_ref, dv_ref,
             g0_ref, g1_ref):
        hn = _gn_block(y_ref[...], s_ref[...], q_ref[...], w_ref[...],
                       b_ref[...], m_ref[...], r_real)
        if relu:
            hn = jnp.maximum(hn, 0.0)
        t = jnp.dot(hn, W_ref[...], preferred_element_type=F32) * dv_ref[...]
        g0_ref[...] = t[:, :16]
        g1_ref[...] = t[:, 16:]

    z2 = lambda i: (0, 0)
    return pl.pallas_call(
        body,
        grid=(R // _BR,),
        in_specs=[pl.BlockSpec((_BR, C), lambda i: (i, 0)),
                  pl.BlockSpec((1, C), z2), pl.BlockSpec((1, C), z2),
                  pl.BlockSpec((1, C), z2), pl.BlockSpec((1, C), z2),
                  pl.BlockSpec((1, C), z2),
                  pl.BlockSpec((C, C), z2),
                  pl.BlockSpec((_BR, C), lambda i: (i, 0))],
        out_specs=[pl.BlockSpec((_BR, 16), lambda i: (i, 0)),
                   pl.BlockSpec((_BR, 16), lambda i: (i, 0))],
        out_shape=[jax.ShapeDtypeStruct((R, 16), F32),
                   jax.ShapeDtypeStruct((R, 16), F32)],
    )(y, S, Q, w2d, b2d, ms2d, W, dinvb)


def _tck_finish(A0, A1, g0, g1, dinvb, bias2d, r_real):
    R = A0.shape[0]

    def body(a0_ref, a1_ref, g0_ref, g1_ref, dv_ref, b_ref,
             y_ref, s_ref, q_ref):
        i = pl.program_id(0)
        t = jnp.concatenate([a0_ref[...] + g0_ref[...],
                             a1_ref[...] + g1_ref[...]], axis=1)
        yb = dv_ref[...] * t + b_ref[...]
        yb = jnp.where(_rowmask(i, r_real), yb, 0.0)
        y_ref[...] = yb

        @pl.when(i == 0)
        def _():
            s_ref[...] = jnp.zeros_like(s_ref)
            q_ref[...] = jnp.zeros_like(q_ref)

        s_ref[...] += jnp.sum(yb, axis=0, keepdims=True)
        q_ref[...] += jnp.sum(yb * yb, axis=0, keepdims=True)

    z2 = lambda i: (0, 0)
    h = lambda i: (i, 0)
    return pl.pallas_call(
        body,
        grid=(R // _BR,),
        in_specs=[pl.BlockSpec((_BR, 16), h), pl.BlockSpec((_BR, 16), h),
                  pl.BlockSpec((_BR, 16), h), pl.BlockSpec((_BR, 16), h),
                  pl.BlockSpec((_BR, C), h), pl.BlockSpec((1, C), z2)],
        out_specs=[pl.BlockSpec((_BR, C), h),
                   pl.BlockSpec((1, C), z2), pl.BlockSpec((1, C), z2)],
        out_shape=[jax.ShapeDtypeStruct((R, C), F32),
                   jax.ShapeDtypeStruct((1, C), F32),
                   jax.ShapeDtypeStruct((1, C), F32)],
    )(A0, A1, g0, g1, dinvb, bias2d)


def _tck_norm(y, S, Q, w2d, b2d, ms2d, r_real):
    R = y.shape[0]

    def body(y_ref, s_ref, q_ref, w_ref, b_ref, m_ref, o_ref):
        o_ref[...] = _gn_block(y_ref[...], s_ref[...], q_ref[...], w_ref[...],
                               b_ref[...], m_ref[...], r_real)

    z2 = lambda i: (0, 0)
    return pl.pallas_call(
        body,
        grid=(R // _BR,),
        in_specs=[pl.BlockSpec((_BR, C), lambda i: (i, 0)),
                  pl.BlockSpec((1, C), z2), pl.BlockSpec((1, C), z2),
                  pl.BlockSpec((1, C), z2), pl.BlockSpec((1, C), z2),
                  pl.BlockSpec((1, C), z2)],
        out_specs=pl.BlockSpec((_BR, C), lambda i: (i, 0)),
        out_shape=jax.ShapeDtypeStruct((R, C), F32),
    )(y, S, Q, w2d, b2d, ms2d)


def _tck_ztable(zl, zr, W3p, W4p, dAb, dBb):
    R = zl.shape[0]

    def body(zl_ref, zr_ref, w3_ref, w4_ref, da_ref, db_ref,
             a0, a1, a2, b0, b1, b2):
        z = zl_ref[...] * zr_ref[...]
        tA = jnp.dot(z, w3_ref[...], preferred_element_type=F32) * da_ref[...]
        tB = jnp.dot(z, w4_ref[...], preferred_element_type=F32) * db_ref[...]
        a0[...] = tA[:, 0:8]
        a1[...] = tA[:, 8:16]
        a2[...] = tA[:, 16:24]
        b0[...] = tB[:, 0:8]
        b1[...] = tB[:, 8:16]
        b2[...] = tB[:, 16:24]

    z2 = lambda i: (0, 0)
    h = lambda i: (i, 0)
    o8 = pl.BlockSpec((_BR, 8), h)
    s8 = jax.ShapeDtypeStruct((R, 8), F32)
    return pl.pallas_call(
        body,
        grid=(R // _BR,),
        in_specs=[pl.BlockSpec((_BR, C), h), pl.BlockSpec((_BR, C), h),
                  pl.BlockSpec((C, C), z2), pl.BlockSpec((C, C), z2),
                  pl.BlockSpec((_BR, C), h), pl.BlockSpec((_BR, C), h)],
        out_specs=[o8, o8, o8, o8, o8, o8],
        out_shape=[s8, s8, s8, s8, s8, s8],
    )(zl, zr, W3p, W4p, dAb, dBb)


def _tck_finishz(A3, g3, dinvb, bias2d, r_real):
    R = A3[0].shape[0]

    def body(a0_ref, a1_ref, a2_ref, g0_ref, g1_ref, g2_ref, dv_ref, b_ref,
             y_ref, s_ref, q_ref):
        i = pl.program_id(0)
        t = jnp.concatenate([a0_ref[...] + g0_ref[...],
                             a1_ref[...] + g1_ref[...],
                             a2_ref[...] + g2_ref[...],
                             jnp.zeros((_BR, 8), F32)], axis=1)
        yb = dv_ref[...] * t + b_ref[...]
        yb = jnp.where(_rowmask(i, r_real), yb, 0.0)
        y_ref[...] = yb

        @pl.when(i == 0)
        def _():
            s_ref[...] = jnp.zeros_like(s_ref)
            q_ref[...] = jnp.zeros_like(q_ref)

        s_ref[...] += jnp.sum(yb, axis=0, keepdims=True)
        q_ref[...] += jnp.sum(yb * yb, axis=0, keepdims=True)

    z2 = lambda i: (0, 0)
    h = lambda i: (i, 0)
    i8 = pl.BlockSpec((_BR, 8), h)
    return pl.pallas_call(
        body,
        grid=(R // _BR,),
        in_specs=[i8, i8, i8, i8, i8, i8,
                  pl.BlockSpec((_BR, C), h), pl.BlockSpec((1, C), z2)],
        out_specs=[pl.BlockSpec((_BR, C), h),
                   pl.BlockSpec((1, C), z2), pl.BlockSpec((1, C), z2)],
        out_shape=[jax.ShapeDtypeStruct((R, C), F32),
                   jax.ShapeDtypeStruct((1, C), F32),
                   jax.ShapeDtypeStruct((1, C), F32)],
    )(A3[0], A3[1], A3[2], g3[0], g3[1], g3[2], dinvb, bias2d)


def _tck_nsum(y3, S3, Q3, p3, y4, S4, Q4, p4, r_real):
    R = y3.shape[0]

    def body(y3_ref, s3_ref, q3_ref, w3_ref, b3_ref, m3_ref,
             y4_ref, s4_ref, q4_ref, w4_ref, b4_ref, m4_ref, o_ref):
        n3 = _gn_block(y3_ref[...], s3_ref[...], q3_ref[...], w3_ref[...],
                       b3_ref[...], m3_ref[...], r_real)
        n4 = _gn_block(y4_ref[...], s4_ref[...], q4_ref[...], w4_ref[...],
                       b4_ref[...], m4_ref[...], r_real)
        o_ref[...] = jnp.maximum(n3, 0.0) + jnp.maximum(n4, 0.0)

    z2 = lambda i: (0, 0)
    h = lambda i: (i, 0)
    c1 = pl.BlockSpec((1, C), z2)
    return pl.pallas_call(
        body,
        grid=(R // _BR,),
        in_specs=[pl.BlockSpec((_BR, C), h), c1, c1, c1, c1, c1,
                  pl.BlockSpec((_BR, C), h), c1, c1, c1, c1, c1],
        out_specs=pl.BlockSpec((_BR, C), h),
        out_shape=jax.ShapeDtypeStruct((R, C), F32),
    )(y3, S3, Q3, p3[0], p3[1], p3[2], y4, S4, Q4, p4[0], p4[1], p4[2])


def _tck_out(sB, wp2d, bp2d):
    def body(s_ref, w_ref, b_ref, o_ref):
        o_ref[...] = jnp.sum(s_ref[...] * w_ref[...], axis=1,
                             keepdims=True) + b_ref[...]

    z2 = lambda i: (0, 0)
    return pl.pallas_call(
        body,
        grid=(pl.cdiv(B, _BR),),
        in_specs=[pl.BlockSpec((_BR, C), lambda i: (i, 0)),
                  pl.BlockSpec((1, C), z2),
                  pl.BlockSpec((1, 1), z2)],
        out_specs=pl.BlockSpec((_BR, 1), lambda i: (i, 0)),
        out_shape=jax.ShapeDtypeStruct((B, 1), F32),
    )(sB, wp2d, bp2d)


def _pad_w(Wm, rows, cols):
    r, c = Wm.shape
    return jnp.pad(Wm, ((0, rows - r), (0, cols - c)))


def _pad_v(v, n, val=0.0):
    return jnp.pad(v, (0, n - v.shape[0]), constant_values=val)


def kernel(x, edge1, pos, idx, ei2, emb, gne_w, gne_b, gne_ms,
           W1, b1, gn1_w, gn1_b, gn1_ms,
           W2, b2, gn2_w, gn2_b, gn2_ms,
           W3, b3, gn3_w, gn3_b, gn3_ms,
           W4, b4, gn4_w, gn4_b, gn4_ms,
           Wp, bp):
    i32 = lambda a: a.astype(I32)
    src1_2d = i32(edge1[0]).reshape(E // 128, 128)
    dst1_2d = i32(edge1[1]).reshape(E // 128, 128)
    e2s_2d = i32(ei2[0]).reshape(E2 // 128, 128)
    e2d_2d = i32(ei2[1]).reshape(E2 // 128, 128)
    pos0 = _pad_v(i32(pos[:, 0]), RP)
    pos1 = _pad_v(i32(pos[:, 1]), RP)
    idxB = _pad_v(i32(idx), RN)
    xi = _pad_v(i32(x), RN)

    zeros6400 = jnp.zeros((6400,), F32)
    ones128 = jnp.ones((128,), F32)
    zeros640x16 = jnp.zeros((640, 16), F32)
    zeros1280x8 = jnp.zeros((1280, 8), F32)

    r2 = lambda v: v.reshape(1, -1)
    gnep = (r2(gne_w), r2(gne_b), r2(gne_ms))
    gn1p = (r2(gn1_w), r2(gn1_b), r2(gn1_ms))
    gn2p = (r2(_pad_v(gn2_w, C, 1.0)), r2(_pad_v(gn2_b, C)),
            r2(_pad_v(gn2_ms, C, 1.0)))
    gn3p = (r2(_pad_v(gn3_w, C, 1.0)), r2(_pad_v(gn3_b, C)),
            r2(_pad_v(gn3_ms, C, 1.0)))
    gn4p = (r2(_pad_v(gn4_w, C, 1.0)), r2(_pad_v(gn4_b, C)),
            r2(_pad_v(gn4_ms, C, 1.0)))
    W2p = _pad_w(W2, C, C)
    W3p = _pad_w(W3, C, C)
    W4p = _pad_w(W4, C, C)
    wp2d = _pad_v(Wp[:, 0], C).reshape(1, C)
    b1p = r2(b1)
    b2p = r2(_pad_v(b2, C))
    b3p = r2(_pad_v(b3, C))
    b4p = r2(_pad_v(b4, C))
    bp2d = bp.reshape(1, 1)

    deg1, degA, degB = _sck_deg(dst1_2d, e2d_2d, e2s_2d, zeros6400, ones128)
    dinv1 = _tck_rsqrt(deg1)
    dinvA = _tck_rsqrt(degA)
    dinvB = _tck_rsqrt(degB)
    dinv1b = jnp.broadcast_to(dinv1[:, None], (RN, C))
    dinvAb = jnp.broadcast_to(dinvA[:, None], (RP, C))
    dinvBb = jnp.broadcast_to(dinvB[:, None], (RP, C))

    h0 = _sck_gather(emb, xi)
    S0, Q0 = _tck_stats(h0, N)

    g1_0, g1_1 = _tck_table(h0, S0, Q0, *gnep, W1, dinv1b, relu=False,
                            r_real=N)
    A1_0, A1_1 = _sck_conv(g1_0, g1_1, src1_2d, dst1_2d, zeros640x16)
    y1, S1, Q1 = _tck_finish(A1_0, A1_1, g1_0, g1_1, dinv1b, b1p, N)

    g2_0, g2_1 = _tck_table(y1, S1, Q1, *gn1p, W2p, dinv1b, relu=True,
                            r_real=N)
    A2_0, A2_1 = _sck_conv(g2_0, g2_1, src1_2d, dst1_2d, zeros640x16)
    y2, S2, Q2 = _tck_finish(A2_0, A2_1, g2_0, g2_1, dinv1b, b2p, N)

    h2 = _tck_norm(y2, S2, Q2, *gn2p, r_real=N)
    zl, zr = _sck_hadamard(h2, pos0, pos1)

    gA0, gA1, gA2, gB0, gB1, gB2 = _tck_ztable(zl, zr, W3p, W4p, dinvAb, dinvBb)
    AA0, AA1, AA2, AB0, AB1, AB2 = _sck_zconv(
        (gA0, gA1, gA2), (gB0, gB1, gB2), e2s_2d, e2d_2d, zeros1280x8)
    y3, S3, Q3 = _tck_finishz((AA0, AA1, AA2), (gA0, gA1, gA2), dinvAb, b3p, P)
    y4, S4, Q4 = _tck_finishz((AB0, AB1, AB2), (gB0, gB1, gB2), dinvBb, b4p, P)

    s = _tck_nsum(y3, S3, Q3, gn3p, y4, S4, Q4, gn4p, P)
    sB = _sck_gather(s, idxB)
    return _tck_out(sB, wp2d, bp2d)


# speedup vs baseline: 17.2934x; 1.3643x over previous
import functools

import jax
import jax.numpy as jnp
from jax import lax
from jax.experimental import pallas as pl
from jax.experimental.pallas import tpu as pltpu
from jax.experimental.pallas import tpu_sc as plsc

N = 100000
E = 1600000
P = 200000
B = 100000
E2 = 800000
C = 32
RN = 102400
RP = 204800

_MESH = plsc.VectorSubcoreMesh(core_axis_name="c", subcore_axis_name="s")
_SC_PARAMS = pltpu.CompilerParams(use_tc_tiling_on_sc=False)
F32 = jnp.float32
I32 = jnp.int32


def _zero_init_1d(zb, acc, size, sid):
    per = size // 16
    base = sid * per
    for j in range(per // 6400):
        pltpu.sync_copy(zb, acc.at[pl.ds(base + j * 6400, 6400)])


def _copy_1d(acc, out, size, sid):
    per = size // 16
    base = sid * per
    for j in range(per // 6400):
        pltpu.sync_copy(acc.at[pl.ds(base + j * 6400, 6400)],
                        out.at[pl.ds(base + j * 6400, 6400)])


def _count_1d(acc, idx2d, nchunks, ob, ib, sid):
    niter = (nchunks + 15) // 16

    def body(i, carry):
        c = i * 16 + sid

        @pl.when(c < nchunks)
        def _():
            pltpu.sync_copy(idx2d.at[c], ib)
            pltpu.sync_copy(ob, acc.at[ib], add=True)

        return carry

    lax.fori_loop(0, niter, body, 0)


def _sck_deg(dst1_2d, e2d_2d, e2s_2d, zeros6400, ones128):
    @functools.partial(
        pl.kernel, mesh=_MESH, compiler_params=_SC_PARAMS,
        out_type=[jax.ShapeDtypeStruct((RN,), F32),
                  jax.ShapeDtypeStruct((RP,), F32),
                  jax.ShapeDtypeStruct((RP,), F32)],
        scratch_types=[pltpu.VMEM_SHARED((RN,), F32),
                       pltpu.VMEM_SHARED((RP,), F32),
                       pltpu.VMEM_SHARED((RP,), F32),
                       pltpu.VMEM((6400,), F32),
                       pltpu.VMEM((128,), F32),
                       pltpu.VMEM((128,), I32)],
    )
    def k(dst1_ref, e2d_ref, e2s_ref, z_ref, o128_ref,
          deg1_ref, degA_ref, degB_ref,
          acc1, accA, accB, zb, ob, ib):
        cid = lax.axis_index("c")
        sid = lax.axis_index("s")
        pltpu.sync_copy(z_ref, zb)
        pltpu.sync_copy(o128_ref, ob)

        @pl.when(cid == 0)
        def _():
            _zero_init_1d(zb, acc1, RN, sid)

        @pl.when(cid == 1)
        def _():
            _zero_init_1d(zb, accA, RP, sid)
            _zero_init_1d(zb, accB, RP, sid)

        plsc.subcore_barrier()

        @pl.when(cid == 0)
        def _():
            _count_1d(acc1, dst1_ref, E // 128, ob, ib, sid)

        @pl.when(cid == 1)
        def _():
            _count_1d(accA, e2d_ref, E2 // 128, ob, ib, sid)
            _count_1d(accB, e2s_ref, E2 // 128, ob, ib, sid)

        plsc.subcore_barrier()

        @pl.when(cid == 0)
        def _():
            _copy_1d(acc1, deg1_ref, RN, sid)

        @pl.when(cid == 1)
        def _():
            _copy_1d(accA, degA_ref, RP, sid)
            _copy_1d(accB, degB_ref, RP, sid)

    return k(dst1_2d, e2d_2d, e2s_2d, zeros6400, ones128)


def _zero_init_rows(zb, acc, nrows, ch, sid):
    per = nrows // 16
    base = sid * per
    for j in range(per // ch):
        pltpu.sync_copy(zb, acc.at[pl.ds(base + j * ch, ch)])


def _copy_rows(acc, out, nrows, ch, sid):
    per = nrows // 16
    base = sid * per
    for j in range(per // ch):
        pltpu.sync_copy(acc.at[pl.ds(base + j * ch, ch)],
                        out.at[pl.ds(base + j * ch, ch)])


def _edge_scan(tbl, acc, sidx2d, didx2d, nchunks, sv, dv, rows, gsems, ssems,
               sid):
    nmfull, tailch = divmod(nchunks, 8)
    nmb = nmfull + (1 if tailch else 0)
    niter = (nmb + 15) // 16

    def macro(m, nch):
        pltpu.sync_copy(sidx2d.at[pl.ds(m * 8, nch)], sv.at[pl.ds(0, nch)])
        pltpu.sync_copy(didx2d.at[pl.ds(m * 8, nch)], dv.at[pl.ds(0, nch)])
        gh = [pltpu.async_copy(tbl.at[sv.at[j]], rows[j], gsems[j])
              for j in range(nch)]
        sh = []
        for j in range(nch):
            gh[j].wait()
            sh.append(pltpu.async_copy(rows[j], acc.at[dv.at[j]], ssems[j],
                                       add=True))
        for h in sh:
            h.wait()

    def body(i, carry):
        m = i * 16 + sid

        @pl.when(m < nmfull)
        def _():
            macro(m, 8)

        if tailch:
            @pl.when(m == nmfull)
            def _():
                macro(m, tailch)

        return carry

    lax.fori_loop(0, niter, body, 0)


def _sck_conv(T0, T1, sidx2d, didx2d, zeros640x16):
    V = T0.shape[0]
    nchunks = sidx2d.shape[0]

    @functools.partial(
        pl.kernel, mesh=_MESH, compiler_params=_SC_PARAMS,
        out_type=[jax.ShapeDtypeStruct((V, 16), F32),
                  jax.ShapeDtypeStruct((V, 16), F32)],
        scratch_types=[pltpu.VMEM_SHARED((V, 16), F32),
                       pltpu.VMEM((640, 16), F32),
                       pltpu.VMEM((8, 128), I32),
                       pltpu.VMEM((8, 128), I32)]
                      + [pltpu.VMEM((128, 16), F32) for _ in range(8)]
                      + [pltpu.SemaphoreType.DMA for _ in range(16)],
    )
    def k(t0_ref, t1_ref, s_ref, d_ref, z_ref, a0_ref, a1_ref,
          acc, zb, sv, dv, *rest):
        rows = rest[0:8]
        gsems = rest[8:16]
        ssems = rest[16:24]
        cid = lax.axis_index("c")
        sid = lax.axis_index("s")
        pltpu.sync_copy(z_ref, zb)
        _zero_init_rows(zb, acc, V, 640, sid)
        plsc.subcore_barrier()

        @pl.when(cid == 0)
        def _():
            _edge_scan(t0_ref, acc, s_ref, d_ref, nchunks, sv, dv, rows,
                       gsems, ssems, sid)

        @pl.when(cid == 1)
        def _():
            _edge_scan(t1_ref, acc, s_ref, d_ref, nchunks, sv, dv, rows,
                       gsems, ssems, sid)

        plsc.subcore_barrier()

        @pl.when(cid == 0)
        def _():
            _copy_rows(acc, a0_ref, V, 640, sid)

        @pl.when(cid == 1)
        def _():
            _copy_rows(acc, a1_ref, V, 640, sid)

    return k(T0, T1, sidx2d, didx2d, zeros640x16)


def _sck_zconv(gA, gB, e2s2d, e2d2d, zeros1280x8):
    nchunks = e2s2d.shape[0]
    out_t = [jax.ShapeDtypeStruct((RP, 8), F32) for _ in range(6)]

    @functools.partial(
        pl.kernel, mesh=_MESH, compiler_params=_SC_PARAMS,
        out_type=out_t,
        scratch_types=[pltpu.VMEM_SHARED((RP, 8), F32),
                       pltpu.VMEM((1280, 8), F32),
                       pltpu.VMEM((8, 128), I32),
                       pltpu.VMEM((8, 128), I32)]
                      + [pltpu.VMEM((128, 8), F32) for _ in range(8)]
                      + [pltpu.SemaphoreType.DMA for _ in range(16)],
    )
    def k(ga0, ga1, ga2, gb0, gb1, gb2, es_ref, ed_ref, z_ref,
          oa0, oa1, oa2, ob0, ob1, ob2,
          acc, zb, sv, dv, *rest):
        rows = rest[0:8]
        gsems = rest[8:16]
        ssems = rest[16:24]
        cid = lax.axis_index("c")
        sid = lax.axis_index("s")
        pltpu.sync_copy(z_ref, zb)
        ga = [ga0, ga1, ga2]
        gb = [gb0, gb1, gb2]
        oa = [oa0, oa1, oa2]
        ob = [ob0, ob1, ob2]
        for r in range(3):
            _zero_init_rows(zb, acc, RP, 1280, sid)
            plsc.subcore_barrier()

            @pl.when(cid == 0)
            def _():
                _edge_scan(ga[r], acc, es_ref, ed_ref, nchunks, sv, dv,
                           rows, gsems, ssems, sid)

            @pl.when(cid == 1)
            def _():
                _edge_scan(gb[r], acc, ed_ref, es_ref, nchunks, sv, dv,
                           rows, gsems, ssems, sid)

            plsc.subcore_barrier()

            @pl.when(cid == 0)
            def _():
                _copy_rows(acc, oa[r], RP, 1280, sid)

            @pl.when(cid == 1)
            def _():
                _copy_rows(acc, ob[r], RP, 1280, sid)

            if r < 2:
                plsc.subcore_barrier()

    return k(gA[0], gA[1], gA[2], gB[0], gB[1], gB[2], e2s2d, e2d2d,
             zeros1280x8)


def _sck_gather(table, idxs):
    Bn = idxs.shape[0]
    nch = Bn // 128
    niter = nch // 32

    @functools.partial(
        pl.kernel, mesh=_MESH, compiler_params=_SC_PARAMS,
        out_type=jax.ShapeDtypeStruct((Bn, 32), F32),
        scratch_types=[pltpu.VMEM((128,), I32),
                       pltpu.VMEM((128, 32), F32),
                       pltpu.SemaphoreType.DMA],
    )
    def k(tbl, idx_ref, out_ref, iv, rows, sem):
        cid = lax.axis_index("c")
        sid = lax.axis_index("s")
        wid = sid * 2 + cid

        def body(i, carry):
            c = i * 32 + wid
            base = c * 128
            pltpu.sync_copy(idx_ref.at[pl.ds(base, 128)], iv)
            pltpu.async_copy(tbl.at[iv], rows, sem).wait()
            pltpu.sync_copy(rows, out_ref.at[pl.ds(base, 128)])
            return carry

        lax.fori_loop(0, niter, body, 0)

    return k(table, idxs)


def _sck_hadamard(table, idxl, idxr):
    Bn = idxl.shape[0]
    nch = Bn // 128
    niter = nch // 32

    @functools.partial(
        pl.kernel, mesh=_MESH, compiler_params=_SC_PARAMS,
        out_type=[jax.ShapeDtypeStruct((Bn, 32), F32),
                  jax.ShapeDtypeStruct((Bn, 32), F32)],
        scratch_types=[pltpu.VMEM((128,), I32),
                       pltpu.VMEM((128,), I32),
                       pltpu.VMEM((128, 32), F32),
                       pltpu.VMEM((128, 32), F32),
                       pltpu.SemaphoreType.DMA,
                       pltpu.SemaphoreType.DMA],
    )
    def k(tbl, il_ref, ir_ref, zl_ref, zr_ref,
          ivl, ivr, rl, rr, sem1, sem2):
        cid = lax.axis_index("c")
        sid = lax.axis_index("s")
        wid = sid * 2 + cid

        def body(i, carry):
            c = i * 32 + wid
            base = c * 128
            pltpu.sync_copy(il_ref.at[pl.ds(base, 128)], ivl)
            pltpu.sync_copy(ir_ref.at[pl.ds(base, 128)], ivr)
            cp1 = pltpu.async_copy(tbl.at[ivl], rl, sem1)
            cp2 = pltpu.async_copy(tbl.at[ivr], rr, sem2)
            cp1.wait()
            cp2.wait()
            pltpu.sync_copy(rl, zl_ref.at[pl.ds(base, 128)])
            pltpu.sync_copy(rr, zr_ref.at[pl.ds(base, 128)])
            return carry

        lax.fori_loop(0, niter, body, 0)

    return k(table, idxl, idxr)


_BR = 2048


def _rowmask(i, r_real):
    gr = i * _BR + lax.broadcasted_iota(I32, (_BR, C), 0)
    return gr < r_real


def _tck_rsqrt(deg):
    R = deg.shape[0]
    BRX = 4096

    def body(d_ref, o_ref):
        o_ref[...] = lax.rsqrt(d_ref[...] + 1.0)

    return pl.pallas_call(
        body,
        grid=(R // BRX,),
        in_specs=[pl.BlockSpec((BRX,), lambda i: (i,))],
        out_specs=pl.BlockSpec((BRX,), lambda i: (i,)),
        out_shape=jax.ShapeDtypeStruct((R,), F32),
    )(deg)


def _tck_stats(y, r_real):
    R = y.shape[0]

    def body(y_ref, s_ref, q_ref):
        i = pl.program_id(0)

        @pl.when(i == 0)
        def _():
            s_ref[...] = jnp.zeros_like(s_ref)
            q_ref[...] = jnp.zeros_like(q_ref)

        yb = jnp.where(_rowmask(i, r_real), y_ref[...], 0.0)
        s_ref[...] += jnp.sum(yb, axis=0, keepdims=True)
        q_ref[...] += jnp.sum(yb * yb, axis=0, keepdims=True)

    return pl.pallas_call(
        body,
        grid=(R // _BR,),
        in_specs=[pl.BlockSpec((_BR, C), lambda i: (i, 0))],
        out_specs=[pl.BlockSpec((1, C), lambda i: (0, 0)),
                   pl.BlockSpec((1, C), lambda i: (0, 0))],
        out_shape=[jax.ShapeDtypeStruct((1, C), F32),
                   jax.ShapeDtypeStruct((1, C), F32)],
    )(y)


def _gn_block(yb, s, q, w, b, ms, r_real):
    mean = s * (1.0 / r_real)
    m2 = q * (1.0 / r_real)
    mm = ms * mean
    var = m2 - 2.0 * mm * mean + mm * mm
    rstd = lax.rsqrt(var + 1e-5)
    return w * (yb - mm) * rstd + b


def _tck_table(y, S, Q, w2d, b2d, ms2d, W, dinvb, relu, r_real):
    R = y.shape[0]

    def body(y_ref, s_ref, q_ref, w_ref, b_ref, m_ref, W_ref, dv_ref,
             g0_ref, g1_ref):
        hn = _gn_block(y_ref[...], s_ref[...], q_ref[...], w_ref[...],
                       b_ref[...], m_ref[...], r_real)
        if relu:
            hn = jnp.maximum(hn, 0.0)
        t = jnp.dot(hn, W_ref[...], preferred_element_type=F32) * dv_ref[...]
        g0_ref[...] = t[:, :16]
        g1_ref[...] = t[:, 16:]

    z2 = lambda i: (0, 0)
    return pl.pallas_call(
        body,
        grid=(R // _BR,),
        in_specs=[pl.BlockSpec((_BR, C), lambda i: (i, 0)),
                  pl.BlockSpec((1, C), z2), pl.BlockSpec((1, C), z2),
                  pl.BlockSpec((1, C), z2), pl.BlockSpec((1, C), z2),
                  pl.BlockSpec((1, C), z2),
                  pl.BlockSpec((C, C), z2),
                  pl.BlockSpec((_BR, C), lambda i: (i, 0))],
        out_specs=[pl.BlockSpec((_BR, 16), lambda i: (i, 0)),
                   pl.BlockSpec((_BR, 16), lambda i: (i, 0))],
        out_shape=[jax.ShapeDtypeStruct((R, 16), F32),
                   jax.ShapeDtypeStruct((R, 16), F32)],
    )(y, S, Q, w2d, b2d, ms2d, W, dinvb)


def _tck_finish(A0, A1, g0, g1, dinvb, bias2d, r_real):
    R = A0.shape[0]

    def body(a0_ref, a1_ref, g0_ref, g1_ref, dv_ref, b_ref,
             y_ref, s_ref, q_ref):
        i = pl.program_id(0)
        t = jnp.concatenate([a0_ref[...] + g0_ref[...],
                             a1_ref[...] + g1_ref[...]], axis=1)
        yb = dv_ref[...] * t + b_ref[...]
        yb = jnp.where(_rowmask(i, r_real), yb, 0.0)
        y_ref[...] = yb

        @pl.when(i == 0)
        def _():
            s_ref[...] = jnp.zeros_like(s_ref)
            q_ref[...] = jnp.zeros_like(q_ref)

        s_ref[...] += jnp.sum(yb, axis=0, keepdims=True)
        q_ref[...] += jnp.sum(yb * yb, axis=0, keepdims=True)

    z2 = lambda i: (0, 0)
    h = lambda i: (i, 0)
    return pl.pallas_call(
        body,
        grid=(R // _BR,),
        in_specs=[pl.BlockSpec((_BR, 16), h), pl.BlockSpec((_BR, 16), h),
                  pl.BlockSpec((_BR, 16), h), pl.BlockSpec((_BR, 16), h),
                  pl.BlockSpec((_BR, C), h), pl.BlockSpec((1, C), z2)],
        out_specs=[pl.BlockSpec((_BR, C), h),
                   pl.BlockSpec((1, C), z2), pl.BlockSpec((1, C), z2)],
        out_shape=[jax.ShapeDtypeStruct((R, C), F32),
                   jax.ShapeDtypeStruct((1, C), F32),
                   jax.ShapeDtypeStruct((1, C), F32)],
    )(A0, A1, g0, g1, dinvb, bias2d)


def _tck_norm(y, S, Q, w2d, b2d, ms2d, r_real):
    R = y.shape[0]

    def body(y_ref, s_ref, q_ref, w_ref, b_ref, m_ref, o_ref):
        o_ref[...] = _gn_block(y_ref[...], s_ref[...], q_ref[...], w_ref[...],
                               b_ref[...], m_ref[...], r_real)

    z2 = lambda i: (0, 0)
    return pl.pallas_call(
        body,
        grid=(R // _BR,),
        in_specs=[pl.BlockSpec((_BR, C), lambda i: (i, 0)),
                  pl.BlockSpec((1, C), z2), pl.BlockSpec((1, C), z2),
                  pl.BlockSpec((1, C), z2), pl.BlockSpec((1, C), z2),
                  pl.BlockSpec((1, C), z2)],
        out_specs=pl.BlockSpec((_BR, C), lambda i: (i, 0)),
        out_shape=jax.ShapeDtypeStruct((R, C), F32),
    )(y, S, Q, w2d, b2d, ms2d)


def _tck_ztable(zl, zr, W3p, W4p, dAb, dBb):
    R = zl.shape[0]

    def body(zl_ref, zr_ref, w3_ref, w4_ref, da_ref, db_ref,
             a0, a1, a2, b0, b1, b2):
        z = zl_ref[...] * zr_ref[...]
        tA = jnp.dot(z, w3_ref[...], preferred_element_type=F32) * da_ref[...]
        tB = jnp.dot(z, w4_ref[...], preferred_element_type=F32) * db_ref[...]
        a0[...] = tA[:, 0:8]
        a1[...] = tA[:, 8:16]
        a2[...] = tA[:, 16:24]
        b0[...] = tB[:, 0:8]
        b1[...] = tB[:, 8:16]
        b2[...] = tB[:, 16:24]

    z2 = lambda i: (0, 0)
    h = lambda i: (i, 0)
    o8 = pl.BlockSpec((_BR, 8), h)
    s8 = jax.ShapeDtypeStruct((R, 8), F32)
    return pl.pallas_call(
        body,
        grid=(R // _BR,),
        in_specs=[pl.BlockSpec((_BR, C), h), pl.BlockSpec((_BR, C), h),
                  pl.BlockSpec((C, C), z2), pl.BlockSpec((C, C), z2),
                  pl.BlockSpec((_BR, C), h), pl.BlockSpec((_BR, C), h)],
        out_specs=[o8, o8, o8, o8, o8, o8],
        out_shape=[s8, s8, s8, s8, s8, s8],
    )(zl, zr, W3p, W4p, dAb, dBb)


def _tck_finishz(A3, g3, dinvb, bias2d, r_real):
    R = A3[0].shape[0]

    def body(a0_ref, a1_ref, a2_ref, g0_ref, g1_ref, g2_ref, dv_ref, b_ref,
             y_ref, s_ref, q_ref):
        i = pl.program_id(0)
        t = jnp.concatenate([a0_ref[...] + g0_ref[...],
                             a1_ref[...] + g1_ref[...],
                             a2_ref[...] + g2_ref[...],
                             jnp.zeros((_BR, 8), F32)], axis=1)
        yb = dv_ref[...] * t + b_ref[...]
        yb = jnp.where(_rowmask(i, r_real), yb, 0.0)
        y_ref[...] = yb

        @pl.when(i == 0)
        def _():
            s_ref[...] = jnp.zeros_like(s_ref)
            q_ref[...] = jnp.zeros_like(q_ref)

        s_ref[...] += jnp.sum(yb, axis=0, keepdims=True)
        q_ref[...] += jnp.sum(yb * yb, axis=0, keepdims=True)

    z2 = lambda i: (0, 0)
    h = lambda i: (i, 0)
    i8 = pl.BlockSpec((_BR, 8), h)
    return pl.pallas_call(
        body,
        grid=(R // _BR,),
        in_specs=[i8, i8, i8, i8, i8, i8,
                  pl.BlockSpec((_BR, C), h), pl.BlockSpec((1, C), z2)],
        out_specs=[pl.BlockSpec((_BR, C), h),
                   pl.BlockSpec((1, C), z2), pl.BlockSpec((1, C), z2)],
        out_shape=[jax.ShapeDtypeStruct((R, C), F32),
                   jax.ShapeDtypeStruct((1, C), F32),
                   jax.ShapeDtypeStruct((1, C), F32)],
    )(A3[0], A3[1], A3[2], g3[0], g3[1], g3[2], dinvb, bias2d)


def _tck_nsum(y3, S3, Q3, p3, y4, S4, Q4, p4, r_real):
    R = y3.shape[0]

    def body(y3_ref, s3_ref, q3_ref, w3_ref, b3_ref, m3_ref,
             y4_ref, s4_ref, q4_ref, w4_ref, b4_ref, m4_ref, o_ref):
        n3 = _gn_block(y3_ref[...], s3_ref[...], q3_ref[...], w3_ref[...],
                       b3_ref[...], m3_ref[...], r_real)
        n4 = _gn_block(y4_ref[...], s4_ref[...], q4_ref[...], w4_ref[...],
                       b4_ref[...], m4_ref[...], r_real)
        o_ref[...] = jnp.maximum(n3, 0.0) + jnp.maximum(n4, 0.0)

    z2 = lambda i: (0, 0)
    h = lambda i: (i, 0)
    c1 = pl.BlockSpec((1, C), z2)
    return pl.pallas_call(
        body,
        grid=(R // _BR,),
        in_specs=[pl.BlockSpec((_BR, C), h), c1, c1, c1, c1, c1,
                  pl.BlockSpec((_BR, C), h), c1, c1, c1, c1, c1],
        out_specs=pl.BlockSpec((_BR, C), h),
        out_shape=jax.ShapeDtypeStruct((R, C), F32),
    )(y3, S3, Q3, p3[0], p3[1], p3[2], y4, S4, Q4, p4[0], p4[1], p4[2])


def _tck_out(sB, wp2d, bp2d):
    def body(s_ref, w_ref, b_ref, o_ref):
        o_ref[...] = jnp.sum(s_ref[...] * w_ref[...], axis=1,
                             keepdims=True) + b_ref[...]

    z2 = lambda i: (0, 0)
    return pl.pallas_call(
        body,
        grid=(pl.cdiv(B, _BR),),
        in_specs=[pl.BlockSpec((_BR, C), lambda i: (i, 0)),
                  pl.BlockSpec((1, C), z2),
                  pl.BlockSpec((1, 1), z2)],
        out_specs=pl.BlockSpec((_BR, 1), lambda i: (i, 0)),
        out_shape=jax.ShapeDtypeStruct((B, 1), F32),
    )(sB, wp2d, bp2d)


def _pad_w(Wm, rows, cols):
    r, c = Wm.shape
    return jnp.pad(Wm, ((0, rows - r), (0, cols - c)))


def _pad_v(v, n, val=0.0):
    return jnp.pad(v, (0, n - v.shape[0]), constant_values=val)


def kernel(x, edge1, pos, idx, ei2, emb, gne_w, gne_b, gne_ms,
           W1, b1, gn1_w, gn1_b, gn1_ms,
           W2, b2, gn2_w, gn2_b, gn2_ms,
           W3, b3, gn3_w, gn3_b, gn3_ms,
           W4, b4, gn4_w, gn4_b, gn4_ms,
           Wp, bp):
    i32 = lambda a: a.astype(I32)
    src1_2d = i32(edge1[0]).reshape(E // 128, 128)
    dst1_2d = i32(edge1[1]).reshape(E // 128, 128)
    e2s_2d = i32(ei2[0]).reshape(E2 // 128, 128)
    e2d_2d = i32(ei2[1]).reshape(E2 // 128, 128)
    pos0 = _pad_v(i32(pos[:, 0]), RP)
    pos1 = _pad_v(i32(pos[:, 1]), RP)
    idxB = _pad_v(i32(idx), RN)
    xi = _pad_v(i32(x), RN)

    zeros6400 = jnp.zeros((6400,), F32)
    ones128 = jnp.ones((128,), F32)
    zeros640x16 = jnp.zeros((640, 16), F32)
    zeros1280x8 = jnp.zeros((1280, 8), F32)

    r2 = lambda v: v.reshape(1, -1)
    gnep = (r2(gne_w), r2(gne_b), r2(gne_ms))
    gn1p = (r2(gn1_w), r2(gn1_b), r2(gn1_ms))
    gn2p = (r2(_pad_v(gn2_w, C, 1.0)), r2(_pad_v(gn2_b, C)),
            r2(_pad_v(gn2_ms, C, 1.0)))
    gn3p = (r2(_pad_v(gn3_w, C, 1.0)), r2(_pad_v(gn3_b, C)),
            r2(_pad_v(gn3_ms, C, 1.0)))
    gn4p = (r2(_pad_v(gn4_w, C, 1.0)), r2(_pad_v(gn4_b, C)),
            r2(_pad_v(gn4_ms, C, 1.0)))
    W2p = _pad_w(W2, C, C)
    W3p = _pad_w(W3, C, C)
    W4p = _pad_w(W4, C, C)
    wp2d = _pad_v(Wp[:, 0], C).reshape(1, C)
    b1p = r2(b1)
    b2p = r2(_pad_v(b2, C))
    b3p = r2(_pad_v(b3, C))
    b4p = r2(_pad_v(b4, C))
    bp2d = bp.reshape(1, 1)

    deg1, degA, degB = _sck_deg(dst1_2d, e2d_2d, e2s_2d, zeros6400, ones128)
    dinv1 = _tck_rsqrt(deg1)
    dinvA = _tck_rsqrt(degA)
    dinvB = _tck_rsqrt(degB)
    dinv1b = jnp.broadcast_to(dinv1[:, None], (RN, C))
    dinvAb = jnp.broadcast_to(dinvA[:, None], (RP, C))
    dinvBb = jnp.broadcast_to(dinvB[:, None], (RP, C))

    h0 = _sck_gather(emb, xi)
    S0, Q0 = _tck_stats(h0, N)

    g1_0, g1_1 = _tck_table(h0, S0, Q0, *gnep, W1, dinv1b, relu=False,
                            r_real=N)
    A1_0, A1_1 = _sck_conv(g1_0, g1_1, src1_2d, dst1_2d, zeros640x16)
    y1, S1, Q1 = _tck_finish(A1_0, A1_1, g1_0, g1_1, dinv1b, b1p, N)

    g2_0, g2_1 = _tck_table(y1, S1, Q1, *gn1p, W2p, dinv1b, relu=True,
                            r_real=N)
    A2_0, A2_1 = _sck_conv(g2_0, g2_1, src1_2d, dst1_2d, zeros640x16)
    y2, S2, Q2 = _tck_finish(A2_0, A2_1, g2_0, g2_1, dinv1b, b2p, N)

    h2 = _tck_norm(y2, S2, Q2, *gn2p, r_real=N)
    zl, zr = _sck_hadamard(h2, pos0, pos1)

    gA0, gA1, gA2, gB0, gB1, gB2 = _tck_ztable(zl, zr, W3p, W4p, dinvAb, dinvBb)
    AA0, AA1, AA2, AB0, AB1, AB2 = _sck_zconv(
        (gA0, gA1, gA2), (gB0, gB1, gB2), e2s_2d, e2d_2d, zeros1280x8)
    y3, S3, Q3 = _tck_finishz((AA0, AA1, AA2), (gA0, gA1, gA2), dinvAb, b3p, P)
    y4, S4, Q4 = _tck_finishz((AB0, AB1, AB2), (gB0, gB1, gB2), dinvBb, b4p, P)

    s = _tck_nsum(y3, S3, Q3, gn3p, y4, S4, Q4, gn4p, P)
    sB = _sck_gather(s, idxB)
    return _tck_out(sB, wp2d, bp2d)


# speedup vs baseline: 19.0696x; 1.1027x over previous
import functools

import jax
import jax.numpy as jnp
from jax import lax
from jax.experimental import pallas as pl
from jax.experimental.pallas import tpu as pltpu
from jax.experimental.pallas import tpu_sc as plsc

N = 100000
E = 1600000
P = 200000
B = 100000
E2 = 800000
C = 32
RN = 102400
RP = 204800

_MESH = plsc.VectorSubcoreMesh(core_axis_name="c", subcore_axis_name="s")
_SC_PARAMS = pltpu.CompilerParams(use_tc_tiling_on_sc=False)
F32 = jnp.float32
I32 = jnp.int32


def _zero_init_1d(zb, acc, size, sid):
    per = size // 16
    base = sid * per
    for j in range(per // 6400):
        pltpu.sync_copy(zb, acc.at[pl.ds(base + j * 6400, 6400)])


def _copy_1d(acc, out, size, sid):
    per = size // 16
    base = sid * per
    for j in range(per // 6400):
        pltpu.sync_copy(acc.at[pl.ds(base + j * 6400, 6400)],
                        out.at[pl.ds(base + j * 6400, 6400)])


def _count_1d(acc, idx2d, nchunks, ob, ibs, isems, ssems, sid):
    niter = ((nchunks + 15) // 16 + 7) // 8

    def body(i, carry):
        cs = [(i * 8 + jj) * 16 + sid for jj in range(8)]
        ih = [None] * 8
        for jj in range(8):
            @pl.when(cs[jj] < nchunks)
            def _(jj=jj):
                ih[jj] = pltpu.async_copy(idx2d.at[cs[jj]], ibs[jj],
                                          isems[jj])
        sh = [None] * 8
        for jj in range(8):
            @pl.when(cs[jj] < nchunks)
            def _(jj=jj):
                ih[jj].wait()
                sh[jj] = pltpu.async_copy(ob, acc.at[ibs[jj]], ssems[jj],
                                          add=True)
        for jj in range(8):
            @pl.when(cs[jj] < nchunks)
            def _(jj=jj):
                sh[jj].wait()

        return carry

    lax.fori_loop(0, niter, body, 0)


def _sck_deg(dst1_2d, e2d_2d, e2s_2d, zeros6400, ones128):
    @functools.partial(
        pl.kernel, mesh=_MESH, compiler_params=_SC_PARAMS,
        out_type=[jax.ShapeDtypeStruct((RN,), F32),
                  jax.ShapeDtypeStruct((RP,), F32),
                  jax.ShapeDtypeStruct((RP,), F32)],
        scratch_types=[pltpu.VMEM_SHARED((RN,), F32),
                       pltpu.VMEM_SHARED((RP,), F32),
                       pltpu.VMEM_SHARED((RP,), F32),
                       pltpu.VMEM((6400,), F32),
                       pltpu.VMEM((128,), F32)]
                      + [pltpu.VMEM((128,), I32) for _ in range(8)]
                      + [pltpu.SemaphoreType.DMA for _ in range(16)],
    )
    def k(dst1_ref, e2d_ref, e2s_ref, z_ref, o128_ref,
          deg1_ref, degA_ref, degB_ref,
          acc1, accA, accB, zb, ob, *rest):
        ibs = rest[0:8]
        isems = rest[8:16]
        ssems = rest[16:24]
        cid = lax.axis_index("c")
        sid = lax.axis_index("s")
        pltpu.sync_copy(z_ref, zb)
        pltpu.sync_copy(o128_ref, ob)

        @pl.when(cid == 0)
        def _():
            _zero_init_1d(zb, acc1, RN, sid)

        @pl.when(cid == 1)
        def _():
            _zero_init_1d(zb, accA, RP, sid)
            _zero_init_1d(zb, accB, RP, sid)

        plsc.subcore_barrier()

        @pl.when(cid == 0)
        def _():
            _count_1d(acc1, dst1_ref, E // 128, ob, ibs, isems, ssems, sid)

        @pl.when(cid == 1)
        def _():
            _count_1d(accA, e2d_ref, E2 // 128, ob, ibs, isems, ssems, sid)
            _count_1d(accB, e2s_ref, E2 // 128, ob, ibs, isems, ssems, sid)

        plsc.subcore_barrier()

        @pl.when(cid == 0)
        def _():
            _copy_1d(acc1, deg1_ref, RN, sid)

        @pl.when(cid == 1)
        def _():
            _copy_1d(accA, degA_ref, RP, sid)
            _copy_1d(accB, degB_ref, RP, sid)

    return k(dst1_2d, e2d_2d, e2s_2d, zeros6400, ones128)


def _zero_init_rows(zb, acc, nrows, ch, sid):
    per = nrows // 16
    base = sid * per
    for j in range(per // ch):
        pltpu.sync_copy(zb, acc.at[pl.ds(base + j * ch, ch)])


def _copy_rows(acc, out, nrows, ch, sid):
    per = nrows // 16
    base = sid * per
    for j in range(per // ch):
        pltpu.sync_copy(acc.at[pl.ds(base + j * ch, ch)],
                        out.at[pl.ds(base + j * ch, ch)])


def _edge_scan(tbl, acc, sidx2d, didx2d, nchunks, sv, dv, rows, gsems, ssems,
               sid):
    nmfull, tailch = divmod(nchunks, 8)
    nmb = nmfull + (1 if tailch else 0)
    niter = (nmb + 15) // 16

    def macro(m, nch):
        pltpu.sync_copy(sidx2d.at[pl.ds(m * 8, nch)], sv.at[pl.ds(0, nch)])
        pltpu.sync_copy(didx2d.at[pl.ds(m * 8, nch)], dv.at[pl.ds(0, nch)])
        gh = [pltpu.async_copy(tbl.at[sv.at[j]], rows[j], gsems[j])
              for j in range(nch)]
        sh = []
        for j in range(nch):
            gh[j].wait()
            sh.append(pltpu.async_copy(rows[j], acc.at[dv.at[j]], ssems[j],
                                       add=True))
        for h in sh:
            h.wait()

    def body(i, carry):
        m = i * 16 + sid

        @pl.when(m < nmfull)
        def _():
            macro(m, 8)

        if tailch:
            @pl.when(m == nmfull)
            def _():
                macro(m, tailch)

        return carry

    lax.fori_loop(0, niter, body, 0)


def _sck_conv(T0, T1, sidx2d, didx2d, zeros640x16):
    V = T0.shape[0]
    nchunks = sidx2d.shape[0]

    @functools.partial(
        pl.kernel, mesh=_MESH, compiler_params=_SC_PARAMS,
        out_type=[jax.ShapeDtypeStruct((V, 16), F32),
                  jax.ShapeDtypeStruct((V, 16), F32)],
        scratch_types=[pltpu.VMEM_SHARED((V, 16), F32),
                       pltpu.VMEM((640, 16), F32),
                       pltpu.VMEM((8, 128), I32),
                       pltpu.VMEM((8, 128), I32)]
                      + [pltpu.VMEM((128, 16), F32) for _ in range(8)]
                      + [pltpu.SemaphoreType.DMA for _ in range(16)],
    )
    def k(t0_ref, t1_ref, s_ref, d_ref, z_ref, a0_ref, a1_ref,
          acc, zb, sv, dv, *rest):
        rows = rest[0:8]
        gsems = rest[8:16]
        ssems = rest[16:24]
        cid = lax.axis_index("c")
        sid = lax.axis_index("s")
        pltpu.sync_copy(z_ref, zb)
        _zero_init_rows(zb, acc, V, 640, sid)
        plsc.subcore_barrier()

        @pl.when(cid == 0)
        def _():
            _edge_scan(t0_ref, acc, s_ref, d_ref, nchunks, sv, dv, rows,
                       gsems, ssems, sid)

        @pl.when(cid == 1)
        def _():
            _edge_scan(t1_ref, acc, s_ref, d_ref, nchunks, sv, dv, rows,
                       gsems, ssems, sid)

        plsc.subcore_barrier()

        @pl.when(cid == 0)
        def _():
            _copy_rows(acc, a0_ref, V, 640, sid)

        @pl.when(cid == 1)
        def _():
            _copy_rows(acc, a1_ref, V, 640, sid)

    return k(T0, T1, sidx2d, didx2d, zeros640x16)


def _sck_zconv(gA, gB, e2s2d, e2d2d, zeros1280x8):
    nchunks = e2s2d.shape[0]
    out_t = [jax.ShapeDtypeStruct((RP, 8), F32) for _ in range(6)]

    @functools.partial(
        pl.kernel, mesh=_MESH, compiler_params=_SC_PARAMS,
        out_type=out_t,
        scratch_types=[pltpu.VMEM_SHARED((RP, 8), F32),
                       pltpu.VMEM((1280, 8), F32),
                       pltpu.VMEM((8, 128), I32),
                       pltpu.VMEM((8, 128), I32)]
                      + [pltpu.VMEM((128, 8), F32) for _ in range(8)]
                      + [pltpu.SemaphoreType.DMA for _ in range(16)],
    )
    def k(ga0, ga1, ga2, gb0, gb1, gb2, es_ref, ed_ref, z_ref,
          oa0, oa1, oa2, ob0, ob1, ob2,
          acc, zb, sv, dv, *rest):
        rows = rest[0:8]
        gsems = rest[8:16]
        ssems = rest[16:24]
        cid = lax.axis_index("c")
        sid = lax.axis_index("s")
        pltpu.sync_copy(z_ref, zb)
        ga = [ga0, ga1, ga2]
        gb = [gb0, gb1, gb2]
        oa = [oa0, oa1, oa2]
        ob = [ob0, ob1, ob2]
        for r in range(3):
            _zero_init_rows(zb, acc, RP, 1280, sid)
            plsc.subcore_barrier()

            @pl.when(cid == 0)
            def _():
                _edge_scan(ga[r], acc, es_ref, ed_ref, nchunks, sv, dv,
                           rows, gsems, ssems, sid)

            @pl.when(cid == 1)
            def _():
                _edge_scan(gb[r], acc, ed_ref, es_ref, nchunks, sv, dv,
                           rows, gsems, ssems, sid)

            plsc.subcore_barrier()

            @pl.when(cid == 0)
            def _():
                _copy_rows(acc, oa[r], RP, 1280, sid)

            @pl.when(cid == 1)
            def _():
                _copy_rows(acc, ob[r], RP, 1280, sid)

            if r < 2:
                plsc.subcore_barrier()

    return k(gA[0], gA[1], gA[2], gB[0], gB[1], gB[2], e2s2d, e2d2d,
             zeros1280x8)


def _sck_gather(table, idxs):
    Bn = idxs.shape[0]
    nch = Bn // 128
    niter = nch // 32

    @functools.partial(
        pl.kernel, mesh=_MESH, compiler_params=_SC_PARAMS,
        out_type=jax.ShapeDtypeStruct((Bn, 32), F32),
        scratch_types=[pltpu.VMEM((128,), I32) for _ in range(8)]
                      + [pltpu.VMEM((128, 32), F32) for _ in range(8)]
                      + [pltpu.SemaphoreType.DMA for _ in range(24)],
    )
    def k(tbl, idx_ref, out_ref, *rest):
        ivs = rest[0:8]
        rows = rest[8:16]
        isems = rest[16:24]
        gsems = rest[24:32]
        osems = rest[32:40]
        cid = lax.axis_index("c")
        sid = lax.axis_index("s")
        wid = sid * 2 + cid

        def body(i, carry):
            ks = [i * 8 + jj for jj in range(8)]
            bases = [(ks[jj] * 32 + wid) * 128 for jj in range(8)]
            ih = [None] * 8
            for jj in range(8):
                @pl.when(ks[jj] < niter)
                def _(jj=jj):
                    ih[jj] = pltpu.async_copy(
                        idx_ref.at[pl.ds(bases[jj], 128)], ivs[jj], isems[jj])
            gh = [None] * 8
            for jj in range(8):
                @pl.when(ks[jj] < niter)
                def _(jj=jj):
                    ih[jj].wait()
                    gh[jj] = pltpu.async_copy(tbl.at[ivs[jj]], rows[jj],
                                              gsems[jj])
            oh = [None] * 8
            for jj in range(8):
                @pl.when(ks[jj] < niter)
                def _(jj=jj):
                    gh[jj].wait()
                    oh[jj] = pltpu.async_copy(
                        rows[jj], out_ref.at[pl.ds(bases[jj], 128)],
                        osems[jj])
            for jj in range(8):
                @pl.when(ks[jj] < niter)
                def _(jj=jj):
                    oh[jj].wait()
            return carry

        lax.fori_loop(0, (niter + 7) // 8, body, 0)

    return k(table, idxs)


def _sck_hadamard(table, idxl, idxr):
    Bn = idxl.shape[0]
    nch = Bn // 128
    niter = nch // 32

    @functools.partial(
        pl.kernel, mesh=_MESH, compiler_params=_SC_PARAMS,
        out_type=[jax.ShapeDtypeStruct((Bn, 32), F32),
                  jax.ShapeDtypeStruct((Bn, 32), F32)],
        scratch_types=[pltpu.VMEM((128,), I32) for _ in range(8)]
                      + [pltpu.VMEM((128, 32), F32) for _ in range(8)]
                      + [pltpu.SemaphoreType.DMA for _ in range(24)],
    )
    def k(tbl, il_ref, ir_ref, zl_ref, zr_ref, *rest):
        ivs = rest[0:8]
        rows = rest[8:16]
        isems = rest[16:24]
        gsems = rest[24:32]
        osems = rest[32:40]
        cid = lax.axis_index("c")
        sid = lax.axis_index("s")
        wid = sid * 2 + cid
        irefs = [il_ref, ir_ref]
        orefs = [zl_ref, zr_ref]

        def body(i, carry):
            ks = [i * 4 + jj // 2 for jj in range(8)]
            bases = [(ks[jj] * 32 + wid) * 128 for jj in range(8)]
            ih = [None] * 8
            for jj in range(8):
                @pl.when(ks[jj] < niter)
                def _(jj=jj):
                    ih[jj] = pltpu.async_copy(
                        irefs[jj % 2].at[pl.ds(bases[jj], 128)], ivs[jj],
                        isems[jj])
            gh = [None] * 8
            for jj in range(8):
                @pl.when(ks[jj] < niter)
                def _(jj=jj):
                    ih[jj].wait()
                    gh[jj] = pltpu.async_copy(tbl.at[ivs[jj]], rows[jj],
                                              gsems[jj])
            oh = [None] * 8
            for jj in range(8):
                @pl.when(ks[jj] < niter)
                def _(jj=jj):
                    gh[jj].wait()
                    oh[jj] = pltpu.async_copy(
                        rows[jj], orefs[jj % 2].at[pl.ds(bases[jj], 128)],
                        osems[jj])
            for jj in range(8):
                @pl.when(ks[jj] < niter)
                def _(jj=jj):
                    oh[jj].wait()
            return carry

        lax.fori_loop(0, (niter + 3) // 4, body, 0)

    return k(table, idxl, idxr)


_BR = 2048


def _rowmask(i, r_real):
    gr = i * _BR + lax.broadcasted_iota(I32, (_BR, C), 0)
    return gr < r_real


def _tck_rsqrt(deg):
    R = deg.shape[0]
    BRX = 4096

    def body(d_ref, o_ref):
        o_ref[...] = lax.rsqrt(d_ref[...] + 1.0)

    return pl.pallas_call(
        body,
        grid=(R // BRX,),
        in_specs=[pl.BlockSpec((BRX,), lambda i: (i,))],
        out_specs=pl.BlockSpec((BRX,), lambda i: (i,)),
        out_shape=jax.ShapeDtypeStruct((R,), F32),
    )(deg)


def _tck_stats(y, r_real):
    R = y.shape[0]

    def body(y_ref, s_ref, q_ref):
        i = pl.program_id(0)

        @pl.when(i == 0)
        def _():
            s_ref[...] = jnp.zeros_like(s_ref)
            q_ref[...] = jnp.zeros_like(q_ref)

        yb = jnp.where(_rowmask(i, r_real), y_ref[...], 0.0)
        s_ref[...] += jnp.sum(yb, axis=0, keepdims=True)
        q_ref[...] += jnp.sum(yb * yb, axis=0, keepdims=True)

    return pl.pallas_call(
        body,
        grid=(R // _BR,),
        in_specs=[pl.BlockSpec((_BR, C), lambda i: (i, 0))],
        out_specs=[pl.BlockSpec((1, C), lambda i: (0, 0)),
                   pl.BlockSpec((1, C), lambda i: (0, 0))],
        out_shape=[jax.ShapeDtypeStruct((1, C), F32),
                   jax.ShapeDtypeStruct((1, C), F32)],
    )(y)


def _gn_block(yb, s, q, w, b, ms, r_real):
    mean = s * (1.0 / r_real)
    m2 = q * (1.0 / r_real)
    mm = ms * mean
    var = m2 - 2.0 * mm * mean + mm * mm
    rstd = lax.rsqrt(var + 1e-5)
    return w * (yb - mm) * rstd + b


def _tck_table(y, S, Q, w2d, b2d, ms2d, W, dinvb, relu, r_real):
    R = y.shape[0]

    def body(y_ref, s_ref, q_ref, w_ref, b_ref, m_ref, W_ref, dv_ref,
             g0_ref, g1_ref):
        hn = _gn_block(y_ref[...], s_ref[...], q_ref[...], w_ref[...],
                       b_ref[...], m_ref[...], r_real)
        if relu:
            hn = jnp.maximum(hn, 0.0)
        t = jnp.dot(hn, W_ref[...], preferred_element_type=F32) * dv_ref[...]
        g0_ref[...] = t[:, :16]
        g1_ref[...] = t[:, 16:]

    z2 = lambda i: (0, 0)
    return pl.pallas_call(
        body,
        grid=(R // _BR,),
        in_specs=[pl.BlockSpec((_BR, C), lambda i: (i, 0)),
                  pl.BlockSpec((1, C), z2), pl.BlockSpec((1, C), z2),
                  pl.BlockSpec((1, C), z2), pl.BlockSpec((1, C), z2),
                  pl.BlockSpec((1, C), z2),
                  pl.BlockSpec((C, C), z2),
                  pl.BlockSpec((_BR, C), lambda i: (i, 0))],
        out_specs=[pl.BlockSpec((_BR, 16), lambda i: (i, 0)),
                   pl.BlockSpec((_BR, 16), lambda i: (i, 0))],
        out_shape=[jax.ShapeDtypeStruct((R, 16), F32),
                   jax.ShapeDtypeStruct((R, 16), F32)],
    )(y, S, Q, w2d, b2d, ms2d, W, dinvb)


def _tck_finish(A0, A1, g0, g1, dinvb, bias2d, r_real):
    R = A0.shape[0]

    def body(a0_ref, a1_ref, g0_ref, g1_ref, dv_ref, b_ref,
             y_ref, s_ref, q_ref):
        i = pl.program_id(0)
        t = jnp.concatenate([a0_ref[...] + g0_ref[...],
                             a1_ref[...] + g1_ref[...]], axis=1)
        yb = dv_ref[...] * t + b_ref[...]
        yb = jnp.where(_rowmask(i, r_real), yb, 0.0)
        y_ref[...] = yb

        @pl.when(i == 0)
        def _():
            s_ref[...] = jnp.zeros_like(s_ref)
            q_ref[...] = jnp.zeros_like(q_ref)

        s_ref[...] += jnp.sum(yb, axis=0, keepdims=True)
        q_ref[...] += jnp.sum(yb * yb, axis=0, keepdims=True)

    z2 = lambda i: (0, 0)
    h = lambda i: (i, 0)
    return pl.pallas_call(
        body,
        grid=(R // _BR,),
        in_specs=[pl.BlockSpec((_BR, 16), h), pl.BlockSpec((_BR, 16), h),
                  pl.BlockSpec((_BR, 16), h), pl.BlockSpec((_BR, 16), h),
                  pl.BlockSpec((_BR, C), h), pl.BlockSpec((1, C), z2)],
        out_specs=[pl.BlockSpec((_BR, C), h),
                   pl.BlockSpec((1, C), z2), pl.BlockSpec((1, C), z2)],
        out_shape=[jax.ShapeDtypeStruct((R, C), F32),
                   jax.ShapeDtypeStruct((1, C), F32),
                   jax.ShapeDtypeStruct((1, C), F32)],
    )(A0, A1, g0, g1, dinvb, bias2d)


def _tck_norm(y, S, Q, w2d, b2d, ms2d, r_real):
    R = y.shape[0]

    def body(y_ref, s_ref, q_ref, w_ref, b_ref, m_ref, o_ref):
        o_ref[...] = _gn_block(y_ref[...], s_ref[...], q_ref[...], w_ref[...],
                               b_ref[...], m_ref[...], r_real)

    z2 = lambda i: (0, 0)
    return pl.pallas_call(
        body,
        grid=(R // _BR,),
        in_specs=[pl.BlockSpec((_BR, C), lambda i: (i, 0)),
                  pl.BlockSpec((1, C), z2), pl.BlockSpec((1, C), z2),
                  pl.BlockSpec((1, C), z2), pl.BlockSpec((1, C), z2),
                  pl.BlockSpec((1, C), z2)],
        out_specs=pl.BlockSpec((_BR, C), lambda i: (i, 0)),
        out_shape=jax.ShapeDtypeStruct((R, C), F32),
    )(y, S, Q, w2d, b2d, ms2d)


def _tck_ztable(zl, zr, W3p, W4p, dAb, dBb):
    R = zl.shape[0]

    def body(zl_ref, zr_ref, w3_ref, w4_ref, da_ref, db_ref,
             a0, a1, a2, b0, b1, b2):
        z = zl_ref[...] * zr_ref[...]
        tA = jnp.dot(z, w3_ref[...], preferred_element_type=F32) * da_ref[...]
        tB = jnp.dot(z, w4_ref[...], preferred_element_type=F32) * db_ref[...]
        a0[...] = tA[:, 0:8]
        a1[...] = tA[:, 8:16]
        a2[...] = tA[:, 16:24]
        b0[...] = tB[:, 0:8]
        b1[...] = tB[:, 8:16]
        b2[...] = tB[:, 16:24]

    z2 = lambda i: (0, 0)
    h = lambda i: (i, 0)
    o8 = pl.BlockSpec((_BR, 8), h)
    s8 = jax.ShapeDtypeStruct((R, 8), F32)
    return pl.pallas_call(
        body,
        grid=(R // _BR,),
        in_specs=[pl.BlockSpec((_BR, C), h), pl.BlockSpec((_BR, C), h),
                  pl.BlockSpec((C, C), z2), pl.BlockSpec((C, C), z2),
                  pl.BlockSpec((_BR, C), h), pl.BlockSpec((_BR, C), h)],
        out_specs=[o8, o8, o8, o8, o8, o8],
        out_shape=[s8, s8, s8, s8, s8, s8],
    )(zl, zr, W3p, W4p, dAb, dBb)


def _tck_finishz(A3, g3, dinvb, bias2d, r_real):
    R = A3[0].shape[0]

    def body(a0_ref, a1_ref, a2_ref, g0_ref, g1_ref, g2_ref, dv_ref, b_ref,
             y_ref, s_ref, q_ref):
        i = pl.program_id(0)
        t = jnp.concatenate([a0_ref[...] + g0_ref[...],
                             a1_ref[...] + g1_ref[...],
                             a2_ref[...] + g2_ref[...],
                             jnp.zeros((_BR, 8), F32)], axis=1)
        yb = dv_ref[...] * t + b_ref[...]
        yb = jnp.where(_rowmask(i, r_real), yb, 0.0)
        y_ref[...] = yb

        @pl.when(i == 0)
        def _():
            s_ref[...] = jnp.zeros_like(s_ref)
            q_ref[...] = jnp.zeros_like(q_ref)

        s_ref[...] += jnp.sum(yb, axis=0, keepdims=True)
        q_ref[...] += jnp.sum(yb * yb, axis=0, keepdims=True)

    z2 = lambda i: (0, 0)
    h = lambda i: (i, 0)
    i8 = pl.BlockSpec((_BR, 8), h)
    return pl.pallas_call(
        body,
        grid=(R // _BR,),
        in_specs=[i8, i8, i8, i8, i8, i8,
                  pl.BlockSpec((_BR, C), h), pl.BlockSpec((1, C), z2)],
        out_specs=[pl.BlockSpec((_BR, C), h),
                   pl.BlockSpec((1, C), z2), pl.BlockSpec((1, C), z2)],
        out_shape=[jax.ShapeDtypeStruct((R, C), F32),
                   jax.ShapeDtypeStruct((1, C), F32),
                   jax.ShapeDtypeStruct((1, C), F32)],
    )(A3[0], A3[1], A3[2], g3[0], g3[1], g3[2], dinvb, bias2d)


def _tck_nsum(y3, S3, Q3, p3, y4, S4, Q4, p4, r_real):
    R = y3.shape[0]

    def body(y3_ref, s3_ref, q3_ref, w3_ref, b3_ref, m3_ref,
             y4_ref, s4_ref, q4_ref, w4_ref, b4_ref, m4_ref, o_ref):
        n3 = _gn_block(y3_ref[...], s3_ref[...], q3_ref[...], w3_ref[...],
                       b3_ref[...], m3_ref[...], r_real)
        n4 = _gn_block(y4_ref[...], s4_ref[...], q4_ref[...], w4_ref[...],
                       b4_ref[...], m4_ref[...], r_real)
        o_ref[...] = jnp.maximum(n3, 0.0) + jnp.maximum(n4, 0.0)

    z2 = lambda i: (0, 0)
    h = lambda i: (i, 0)
    c1 = pl.BlockSpec((1, C), z2)
    return pl.pallas_call(
        body,
        grid=(R // _BR,),
        in_specs=[pl.BlockSpec((_BR, C), h), c1, c1, c1, c1, c1,
                  pl.BlockSpec((_BR, C), h), c1, c1, c1, c1, c1],
        out_specs=pl.BlockSpec((_BR, C), h),
        out_shape=jax.ShapeDtypeStruct((R, C), F32),
    )(y3, S3, Q3, p3[0], p3[1], p3[2], y4, S4, Q4, p4[0], p4[1], p4[2])


def _tck_out(sB, wp2d, bp2d):
    def body(s_ref, w_ref, b_ref, o_ref):
        o_ref[...] = jnp.sum(s_ref[...] * w_ref[...], axis=1,
                             keepdims=True) + b_ref[...]

    z2 = lambda i: (0, 0)
    return pl.pallas_call(
        body,
        grid=(pl.cdiv(B, _BR),),
        in_specs=[pl.BlockSpec((_BR, C), lambda i: (i, 0)),
                  pl.BlockSpec((1, C), z2),
                  pl.BlockSpec((1, 1), z2)],
        out_specs=pl.BlockSpec((_BR, 1), lambda i: (i, 0)),
        out_shape=jax.ShapeDtypeStruct((B, 1), F32),
    )(sB, wp2d, bp2d)


def _pad_w(Wm, rows, cols):
    r, c = Wm.shape
    return jnp.pad(Wm, ((0, rows - r), (0, cols - c)))


def _pad_v(v, n, val=0.0):
    return jnp.pad(v, (0, n - v.shape[0]), constant_values=val)


def kernel(x, edge1, pos, idx, ei2, emb, gne_w, gne_b, gne_ms,
           W1, b1, gn1_w, gn1_b, gn1_ms,
           W2, b2, gn2_w, gn2_b, gn2_ms,
           W3, b3, gn3_w, gn3_b, gn3_ms,
           W4, b4, gn4_w, gn4_b, gn4_ms,
           Wp, bp):
    i32 = lambda a: a.astype(I32)
    src1_2d = i32(edge1[0]).reshape(E // 128, 128)
    dst1_2d = i32(edge1[1]).reshape(E // 128, 128)
    e2s_2d = i32(ei2[0]).reshape(E2 // 128, 128)
    e2d_2d = i32(ei2[1]).reshape(E2 // 128, 128)
    pos0 = _pad_v(i32(pos[:, 0]), RP)
    pos1 = _pad_v(i32(pos[:, 1]), RP)
    idxB = _pad_v(i32(idx), RN)
    xi = _pad_v(i32(x), RN)

    zeros6400 = jnp.zeros((6400,), F32)
    ones128 = jnp.ones((128,), F32)
    zeros640x16 = jnp.zeros((640, 16), F32)
    zeros1280x8 = jnp.zeros((1280, 8), F32)

    r2 = lambda v: v.reshape(1, -1)
    gnep = (r2(gne_w), r2(gne_b), r2(gne_ms))
    gn1p = (r2(gn1_w), r2(gn1_b), r2(gn1_ms))
    gn2p = (r2(_pad_v(gn2_w, C, 1.0)), r2(_pad_v(gn2_b, C)),
            r2(_pad_v(gn2_ms, C, 1.0)))
    gn3p = (r2(_pad_v(gn3_w, C, 1.0)), r2(_pad_v(gn3_b, C)),
            r2(_pad_v(gn3_ms, C, 1.0)))
    gn4p = (r2(_pad_v(gn4_w, C, 1.0)), r2(_pad_v(gn4_b, C)),
            r2(_pad_v(gn4_ms, C, 1.0)))
    W2p = _pad_w(W2, C, C)
    W3p = _pad_w(W3, C, C)
    W4p = _pad_w(W4, C, C)
    wp2d = _pad_v(Wp[:, 0], C).reshape(1, C)
    b1p = r2(b1)
    b2p = r2(_pad_v(b2, C))
    b3p = r2(_pad_v(b3, C))
    b4p = r2(_pad_v(b4, C))
    bp2d = bp.reshape(1, 1)

    deg1, degA, degB = _sck_deg(dst1_2d, e2d_2d, e2s_2d, zeros6400, ones128)
    dinv1 = _tck_rsqrt(deg1)
    dinvA = _tck_rsqrt(degA)
    dinvB = _tck_rsqrt(degB)
    dinv1b = jnp.broadcast_to(dinv1[:, None], (RN, C))
    dinvAb = jnp.broadcast_to(dinvA[:, None], (RP, C))
    dinvBb = jnp.broadcast_to(dinvB[:, None], (RP, C))

    h0 = _sck_gather(emb, xi)
    S0, Q0 = _tck_stats(h0, N)

    g1_0, g1_1 = _tck_table(h0, S0, Q0, *gnep, W1, dinv1b, relu=False,
                            r_real=N)
    A1_0, A1_1 = _sck_conv(g1_0, g1_1, src1_2d, dst1_2d, zeros640x16)
    y1, S1, Q1 = _tck_finish(A1_0, A1_1, g1_0, g1_1, dinv1b, b1p, N)

    g2_0, g2_1 = _tck_table(y1, S1, Q1, *gn1p, W2p, dinv1b, relu=True,
                            r_real=N)
    A2_0, A2_1 = _sck_conv(g2_0, g2_1, src1_2d, dst1_2d, zeros640x16)
    y2, S2, Q2 = _tck_finish(A2_0, A2_1, g2_0, g2_1, dinv1b, b2p, N)

    h2 = _tck_norm(y2, S2, Q2, *gn2p, r_real=N)
    zl, zr = _sck_hadamard(h2, pos0, pos1)

    gA0, gA1, gA2, gB0, gB1, gB2 = _tck_ztable(zl, zr, W3p, W4p, dinvAb, dinvBb)
    AA0, AA1, AA2, AB0, AB1, AB2 = _sck_zconv(
        (gA0, gA1, gA2), (gB0, gB1, gB2), e2s_2d, e2d_2d, zeros1280x8)
    y3, S3, Q3 = _tck_finishz((AA0, AA1, AA2), (gA0, gA1, gA2), dinvAb, b3p, P)
    y4, S4, Q4 = _tck_finishz((AB0, AB1, AB2), (gB0, gB1, gB2), dinvBb, b4p, P)

    s = _tck_nsum(y3, S3, Q3, gn3p, y4, S4, Q4, gn4p, P)
    sB = _sck_gather(s, idxB)
    return _tck_out(sB, wp2d, bp2d)
